# baseline (speedup 1.0000x reference)
import functools

import jax
import jax.numpy as jnp
from jax import lax
from jax.experimental import pallas as pl
from jax.experimental.pallas import tpu as pltpu
from jax.experimental.pallas import tpu_sc as plsc

B = 16384
D = 32
GROUPS = 4
WIDE = D * GROUPS
NC = 2
NS = 16
NW = NC * NS
B_PER_W = B // NW
IDX_CHUNK = 128
N_CHUNKS = B_PER_W // IDX_CHUNK
L = 16


def _make_sc_gather():
    mesh = plsc.VectorSubcoreMesh(core_axis_name="c", subcore_axis_name="s")

    @functools.partial(
        pl.kernel,
        out_type=(
            jax.ShapeDtypeStruct((B, WIDE), jnp.float32),
            jax.ShapeDtypeStruct((B, WIDE), jnp.float32),
        ),
        mesh=mesh,
        scratch_types=[
            pltpu.VMEM((B_PER_W,), jnp.int32),
            pltpu.VMEM((B_PER_W,), jnp.int32),
            pltpu.VMEM((B_PER_W,), jnp.int32),
            pltpu.VMEM((B_PER_W, WIDE), jnp.float32),
            pltpu.SemaphoreType.DMA,
        ],
    )
    def gather(ut_hbm, pt_hbm, ui_hbm, pi_hbm, uo_hbm, po_hbm,
               uidx_v, pidx_v, sidx_v, rows_v, sem):
        wid = lax.axis_index("s") * NC + lax.axis_index("c")
        base = wid * B_PER_W
        pltpu.sync_copy(ui_hbm.at[pl.ds(base, B_PER_W)], uidx_v)
        pltpu.sync_copy(pi_hbm.at[pl.ds(base, B_PER_W)], pidx_v)

        def run_table(idx_v, t_hbm, o_hbm):
            for i in range(B_PER_W // L):
                sl = pl.ds(i * L, L)
                sidx_v[sl] = lax.shift_right_logical(idx_v[sl], 2)
            copies = []
            for j in range(N_CHUNKS):
                sl = pl.ds(j * IDX_CHUNK, IDX_CHUNK)
                copies.append(pltpu.async_copy(
                    t_hbm.at[sidx_v.at[sl]], rows_v.at[sl], sem))
            for c in copies:
                c.wait()
            pltpu.sync_copy(rows_v, o_hbm.at[pl.ds(base, B_PER_W)])

        run_table(uidx_v, ut_hbm, uo_hbm)
        run_table(pidx_v, pt_hbm, po_hbm)

    return gather


_sc_gather = _make_sc_gather()

BLK = 1024


def _mlp_body(uf_ref, pf_ref, uix_ref, pix_ref, w1u_ref, w1p_ref, b1_ref,
              w2_ref, b2_ref, w3_ref, b3_ref, o_ref):
    colgrp = lax.broadcasted_iota(jnp.int32, (1, WIDE), 1) // D
    um = jnp.where(colgrp == (uix_ref[...] & (GROUPS - 1)), uf_ref[...], 0.0)
    pm = jnp.where(colgrp == (pix_ref[...] & (GROUPS - 1)), pf_ref[...], 0.0)
    h1 = (jnp.dot(um, w1u_ref[...], preferred_element_type=jnp.float32)
          + jnp.dot(pm, w1p_ref[...], preferred_element_type=jnp.float32)
          + b1_ref[...])
    h1 = jnp.maximum(h1, 0.0)
    h2 = jnp.dot(h1, w2_ref[...], preferred_element_type=jnp.float32) + b2_ref[...]
    h2 = jnp.maximum(h2, 0.0)
    o_ref[...] = jnp.sum(h2 * w3_ref[...], axis=1) + b3_ref[0, 0]


def _mlp(uf, pf, uix, pix, w1u4, w1p4, b1, w2, b2, w3, b3):
    full = lambda i: (0, 0)
    return pl.pallas_call(
        _mlp_body,
        out_shape=jax.ShapeDtypeStruct((B,), jnp.float32),
        grid=(B // BLK,),
        in_specs=[
            pl.BlockSpec((BLK, WIDE), lambda i: (i, 0)),
            pl.BlockSpec((BLK, WIDE), lambda i: (i, 0)),
            pl.BlockSpec((BLK, 1), lambda i: (i, 0)),
            pl.BlockSpec((BLK, 1), lambda i: (i, 0)),
            pl.BlockSpec((WIDE, 64), full),
            pl.BlockSpec((WIDE, 64), full),
            pl.BlockSpec((1, 64), full),
            pl.BlockSpec((64, 32), full),
            pl.BlockSpec((1, 32), full),
            pl.BlockSpec((1, 32), full),
            pl.BlockSpec((1, 1), full),
        ],
        out_specs=pl.BlockSpec((BLK,), lambda i: (i,)),
    )(uf, pf, uix, pix, w1u4, w1p4, b1, w2, b2, w3, b3)


def kernel(user_tensor, product_tensor, user_table, product_table,
           W1, b1, W2, b2, W3, b3):
    uix = user_tensor.astype(jnp.int32)
    pix = product_tensor.astype(jnp.int32)
    ut_wide = user_table.reshape(-1, WIDE)
    pt_wide = product_table.reshape(-1, WIDE)
    uf, pf = _sc_gather(ut_wide, pt_wide, uix, pix)
    w1u = W1[:, :D].T
    w1p = W1[:, D:].T
    w1u4 = jnp.concatenate([w1u] * GROUPS, axis=0)
    w1p4 = jnp.concatenate([w1p] * GROUPS, axis=0)
    return _mlp(uf, pf, uix.reshape(B, 1), pix.reshape(B, 1),
                w1u4, w1p4, b1.reshape(1, 64), W2.T, b2.reshape(1, 32),
                W3.reshape(1, 32), b3.reshape(1, 1))


# speedup vs baseline: 1.4399x; 1.4399x over previous
import functools

import jax
import jax.numpy as jnp
from jax import lax
from jax.experimental import pallas as pl
from jax.experimental.pallas import tpu as pltpu
from jax.experimental.pallas import tpu_sc as plsc

B = 16384
D = 32
N_ROWS = 1000000
GROUPS = 4
WIDE = D * GROUPS
PANEL = 2048
SEG = PANEL // GROUPS
N_PANELS = -(-N_ROWS // PANEL)
N_WIDE = N_PANELS * SEG
NC = 2
NS = 16
NW = NC * NS
B_PER_W = B // NW
IDX_CHUNK = 128
N_CHUNKS = B_PER_W // IDX_CHUNK
L = 16


def _pack_body(u_ref, p_ref, uo_ref, po_ref):
    for src, dst in ((u_ref, uo_ref), (p_ref, po_ref)):
        x = src[...]
        dst[...] = jnp.concatenate(
            [jnp.transpose(x[:, g * SEG:(g + 1) * SEG]) for g in range(GROUPS)],
            axis=1)


def _pack(ut, pt):
    return pl.pallas_call(
        _pack_body,
        out_shape=(
            jax.ShapeDtypeStruct((N_WIDE, WIDE), jnp.float32),
            jax.ShapeDtypeStruct((N_WIDE, WIDE), jnp.float32),
        ),
        grid=(N_PANELS,),
        in_specs=[
            pl.BlockSpec((D, PANEL), lambda i: (0, i)),
            pl.BlockSpec((D, PANEL), lambda i: (0, i)),
        ],
        out_specs=(
            pl.BlockSpec((SEG, WIDE), lambda i: (i, 0)),
            pl.BlockSpec((SEG, WIDE), lambda i: (i, 0)),
        ),
    )(ut, pt)


def _make_sc_gather():
    mesh = plsc.VectorSubcoreMesh(core_axis_name="c", subcore_axis_name="s")

    @functools.partial(
        pl.kernel,
        out_type=(
            jax.ShapeDtypeStruct((B, WIDE), jnp.float32),
            jax.ShapeDtypeStruct((B, WIDE), jnp.float32),
        ),
        mesh=mesh,
        scratch_types=[
            pltpu.VMEM((B_PER_W,), jnp.int32),
            pltpu.VMEM((B_PER_W,), jnp.int32),
            pltpu.VMEM((B_PER_W,), jnp.int32),
            pltpu.VMEM((B_PER_W, WIDE), jnp.float32),
            pltpu.SemaphoreType.DMA,
        ],
    )
    def gather(ut_hbm, pt_hbm, ui_hbm, pi_hbm, uo_hbm, po_hbm,
               uidx_v, pidx_v, sidx_v, rows_v, sem):
        wid = lax.axis_index("s") * NC + lax.axis_index("c")
        base = wid * B_PER_W
        pltpu.sync_copy(ui_hbm.at[pl.ds(base, B_PER_W)], uidx_v)
        pltpu.sync_copy(pi_hbm.at[pl.ds(base, B_PER_W)], pidx_v)

        def run_table(idx_v, t_hbm, o_hbm):
            for i in range(B_PER_W // L):
                sl = pl.ds(i * L, L)
                v = idx_v[sl]
                sidx_v[sl] = (lax.shift_right_logical(v, 11) * SEG
                              + lax.bitwise_and(v, SEG - 1))
            copies = []
            for j in range(N_CHUNKS):
                sl = pl.ds(j * IDX_CHUNK, IDX_CHUNK)
                copies.append(pltpu.async_copy(
                    t_hbm.at[sidx_v.at[sl]], rows_v.at[sl], sem))
            for c in copies:
                c.wait()
            pltpu.sync_copy(rows_v, o_hbm.at[pl.ds(base, B_PER_W)])

        run_table(uidx_v, ut_hbm, uo_hbm)
        run_table(pidx_v, pt_hbm, po_hbm)

    return gather


_sc_gather = _make_sc_gather()

BLK = 1024


def _mlp_body(uf_ref, pf_ref, uix_ref, pix_ref, w1u_ref, w1p_ref, b1_ref,
              w2_ref, b2_ref, w3_ref, b3_ref, o_ref):
    colgrp = lax.broadcasted_iota(jnp.int32, (1, WIDE), 1) // D
    usel = lax.bitwise_and(lax.shift_right_logical(uix_ref[...], 9), 3)
    psel = lax.bitwise_and(lax.shift_right_logical(pix_ref[...], 9), 3)
    um = jnp.where(colgrp == usel, uf_ref[...], 0.0)
    pm = jnp.where(colgrp == psel, pf_ref[...], 0.0)
    h1 = (jnp.dot(um, w1u_ref[...], preferred_element_type=jnp.float32)
          + jnp.dot(pm, w1p_ref[...], preferred_element_type=jnp.float32)
          + b1_ref[...])
    h1 = jnp.maximum(h1, 0.0)
    h2 = jnp.dot(h1, w2_ref[...], preferred_element_type=jnp.float32) + b2_ref[...]
    h2 = jnp.maximum(h2, 0.0)
    o_ref[...] = jnp.sum(h2 * w3_ref[...], axis=1) + b3_ref[0, 0]


def _mlp(uf, pf, uix, pix, w1u4, w1p4, b1, w2, b2, w3, b3):
    full = lambda i: (0, 0)
    return pl.pallas_call(
        _mlp_body,
        out_shape=jax.ShapeDtypeStruct((B,), jnp.float32),
        grid=(B // BLK,),
        in_specs=[
            pl.BlockSpec((BLK, WIDE), lambda i: (i, 0)),
            pl.BlockSpec((BLK, WIDE), lambda i: (i, 0)),
            pl.BlockSpec((BLK, 1), lambda i: (i, 0)),
            pl.BlockSpec((BLK, 1), lambda i: (i, 0)),
            pl.BlockSpec((WIDE, 64), full),
            pl.BlockSpec((WIDE, 64), full),
            pl.BlockSpec((1, 64), full),
            pl.BlockSpec((64, 32), full),
            pl.BlockSpec((1, 32), full),
            pl.BlockSpec((1, 32), full),
            pl.BlockSpec((1, 1), full),
        ],
        out_specs=pl.BlockSpec((BLK,), lambda i: (i,)),
    )(uf, pf, uix, pix, w1u4, w1p4, b1, w2, b2, w3, b3)


def kernel(user_tensor, product_tensor, user_table, product_table,
           W1, b1, W2, b2, W3, b3):
    uix = user_tensor.astype(jnp.int32)
    pix = product_tensor.astype(jnp.int32)
    u_wide, p_wide = _pack(user_table.T, product_table.T)
    uf, pf = _sc_gather(u_wide, p_wide, uix, pix)
    w1u = W1[:, :D].T
    w1p = W1[:, D:].T
    w1u4 = jnp.concatenate([w1u] * GROUPS, axis=0)
    w1p4 = jnp.concatenate([w1p] * GROUPS, axis=0)
    return _mlp(uf, pf, uix.reshape(B, 1), pix.reshape(B, 1),
                w1u4, w1p4, b1.reshape(1, 64), W2.T, b2.reshape(1, 32),
                W3.reshape(1, 32), b3.reshape(1, 1))


# speedup vs baseline: 2.1345x; 1.4824x over previous
import functools

import jax
import jax.numpy as jnp
from jax import lax
from jax.experimental import pallas as pl
from jax.experimental.pallas import tpu as pltpu
from jax.experimental.pallas import tpu_sc as plsc

B = 16384
D = 32
N_ROWS = 1000000
GROUPS = 4
WIDE = D * GROUPS
PANEL = 2048
SEG = PANEL // GROUPS
N_PANELS = -(-N_ROWS // PANEL)
N_WIDE = N_PANELS * SEG
NC = 2
NS = 16
NW = NC * NS
B_PER_W = B // NW
IDX_CHUNK = 128
N_CHUNKS = B_PER_W // IDX_CHUNK
L = 16


def _pack_body(u_ref, p_ref, uo_ref, po_ref):
    for src, dst in ((u_ref, uo_ref), (p_ref, po_ref)):
        x = src[...]
        stacked = jnp.concatenate(
            [x[:, g * SEG:(g + 1) * SEG] for g in range(GROUPS)], axis=0)
        dst[...] = jnp.transpose(stacked)


def _pack(ut, pt):
    return pl.pallas_call(
        _pack_body,
        out_shape=(
            jax.ShapeDtypeStruct((N_WIDE, WIDE), jnp.float32),
            jax.ShapeDtypeStruct((N_WIDE, WIDE), jnp.float32),
        ),
        grid=(N_PANELS,),
        in_specs=[
            pl.BlockSpec((D, PANEL), lambda i: (0, i)),
            pl.BlockSpec((D, PANEL), lambda i: (0, i)),
        ],
        out_specs=(
            pl.BlockSpec((SEG, WIDE), lambda i: (i, 0)),
            pl.BlockSpec((SEG, WIDE), lambda i: (i, 0)),
        ),
    )(ut, pt)


def _make_sc_gather():
    mesh = plsc.VectorSubcoreMesh(core_axis_name="c", subcore_axis_name="s")

    @functools.partial(
        pl.kernel,
        out_type=(
            jax.ShapeDtypeStruct((B, WIDE), jnp.float32),
            jax.ShapeDtypeStruct((B, WIDE), jnp.float32),
        ),
        mesh=mesh,
        scratch_types=[
            pltpu.VMEM((B_PER_W,), jnp.int32),
            pltpu.VMEM((B_PER_W,), jnp.int32),
            pltpu.VMEM((B_PER_W,), jnp.int32),
            pltpu.VMEM((B_PER_W, WIDE), jnp.float32),
            pltpu.SemaphoreType.DMA,
        ],
    )
    def gather(ut_hbm, pt_hbm, ui_hbm, pi_hbm, uo_hbm, po_hbm,
               uidx_v, pidx_v, sidx_v, rows_v, sem):
        wid = lax.axis_index("s") * NC + lax.axis_index("c")
        base = wid * B_PER_W
        pltpu.sync_copy(ui_hbm.at[pl.ds(base, B_PER_W)], uidx_v)
        pltpu.sync_copy(pi_hbm.at[pl.ds(base, B_PER_W)], pidx_v)

        def run_table(idx_v, t_hbm, o_hbm):
            for i in range(B_PER_W // L):
                sl = pl.ds(i * L, L)
                v = idx_v[sl]
                sidx_v[sl] = (lax.shift_right_logical(v, 11) * SEG
                              + lax.bitwise_and(v, SEG - 1))
            copies = []
            for j in range(N_CHUNKS):
                sl = pl.ds(j * IDX_CHUNK, IDX_CHUNK)
                copies.append(pltpu.async_copy(
                    t_hbm.at[sidx_v.at[sl]], rows_v.at[sl], sem))
            for c in copies:
                c.wait()
            pltpu.sync_copy(rows_v, o_hbm.at[pl.ds(base, B_PER_W)])

        run_table(uidx_v, ut_hbm, uo_hbm)
        run_table(pidx_v, pt_hbm, po_hbm)

    return gather


_sc_gather = _make_sc_gather()

BLK = 1024


def _mlp_body(uf_ref, pf_ref, uix_ref, pix_ref, w1u_ref, w1p_ref, b1_ref,
              w2_ref, b2_ref, w3_ref, b3_ref, o_ref):
    colgrp = lax.broadcasted_iota(jnp.int32, (1, WIDE), 1) // D
    usel = lax.bitwise_and(lax.shift_right_logical(uix_ref[...], 9), 3)
    psel = lax.bitwise_and(lax.shift_right_logical(pix_ref[...], 9), 3)
    um = jnp.where(colgrp == usel, uf_ref[...], 0.0)
    pm = jnp.where(colgrp == psel, pf_ref[...], 0.0)
    h1 = (jnp.dot(um, w1u_ref[...], preferred_element_type=jnp.float32)
          + jnp.dot(pm, w1p_ref[...], preferred_element_type=jnp.float32)
          + b1_ref[...])
    h1 = jnp.maximum(h1, 0.0)
    h2 = jnp.dot(h1, w2_ref[...], preferred_element_type=jnp.float32) + b2_ref[...]
    h2 = jnp.maximum(h2, 0.0)
    o_ref[...] = jnp.sum(h2 * w3_ref[...], axis=1) + b3_ref[0, 0]


def _mlp(uf, pf, uix, pix, w1u4, w1p4, b1, w2, b2, w3, b3):
    full = lambda i: (0, 0)
    return pl.pallas_call(
        _mlp_body,
        out_shape=jax.ShapeDtypeStruct((B,), jnp.float32),
        grid=(B // BLK,),
        in_specs=[
            pl.BlockSpec((BLK, WIDE), lambda i: (i, 0)),
            pl.BlockSpec((BLK, WIDE), lambda i: (i, 0)),
            pl.BlockSpec((BLK, 1), lambda i: (i, 0)),
            pl.BlockSpec((BLK, 1), lambda i: (i, 0)),
            pl.BlockSpec((WIDE, 64), full),
            pl.BlockSpec((WIDE, 64), full),
            pl.BlockSpec((1, 64), full),
            pl.BlockSpec((64, 32), full),
            pl.BlockSpec((1, 32), full),
            pl.BlockSpec((1, 32), full),
            pl.BlockSpec((1, 1), full),
        ],
        out_specs=pl.BlockSpec((BLK,), lambda i: (i,)),
    )(uf, pf, uix, pix, w1u4, w1p4, b1, w2, b2, w3, b3)


def kernel(user_tensor, product_tensor, user_table, product_table,
           W1, b1, W2, b2, W3, b3):
    uix = user_tensor.astype(jnp.int32)
    pix = product_tensor.astype(jnp.int32)
    u_wide, p_wide = _pack(user_table.T, product_table.T)
    uf, pf = _sc_gather(u_wide, p_wide, uix, pix)
    w1u = W1[:, :D].T
    w1p = W1[:, D:].T
    w1u4 = jnp.concatenate([w1u] * GROUPS, axis=0)
    w1p4 = jnp.concatenate([w1p] * GROUPS, axis=0)
    return _mlp(uf, pf, uix.reshape(B, 1), pix.reshape(B, 1),
                w1u4, w1p4, b1.reshape(1, 64), W2.T, b2.reshape(1, 32),
                W3.reshape(1, 32), b3.reshape(1, 1))


# speedup vs baseline: 3.6234x; 1.6975x over previous
import functools

import jax
import jax.numpy as jnp
from jax import lax
from jax.experimental import pallas as pl
from jax.experimental.pallas import tpu as pltpu
from jax.experimental.pallas import tpu_sc as plsc

B = 16384
D = 32
N_ROWS = 1000000
GROUPS = 4
WIDE = D * GROUPS
PANEL = 8192
SEG = PANEL // GROUPS
SH_PANEL = PANEL.bit_length() - 1
SH_SEG = SEG.bit_length() - 1
N_PANELS = -(-N_ROWS // PANEL)
N_WIDE = N_PANELS * SEG
NC = 2
NS = 16
NW = NC * NS
B_PER_W = B // NW
IDX_CHUNK = 128
N_CHUNKS = B_PER_W // IDX_CHUNK
L = 16


def _pack_body(u_ref, p_ref, uo_ref, po_ref):
    for src, dst in ((u_ref, uo_ref), (p_ref, po_ref)):
        x = src[...]
        stacked = jnp.concatenate(
            [x[:, g * SEG:(g + 1) * SEG] for g in range(GROUPS)], axis=0)
        dst[...] = jnp.transpose(stacked)


def _pack(ut, pt):
    return pl.pallas_call(
        _pack_body,
        out_shape=(
            jax.ShapeDtypeStruct((N_WIDE, WIDE), jnp.float32),
            jax.ShapeDtypeStruct((N_WIDE, WIDE), jnp.float32),
        ),
        grid=(N_PANELS,),
        in_specs=[
            pl.BlockSpec((D, PANEL), lambda i: (0, i)),
            pl.BlockSpec((D, PANEL), lambda i: (0, i)),
        ],
        out_specs=(
            pl.BlockSpec((SEG, WIDE), lambda i: (i, 0)),
            pl.BlockSpec((SEG, WIDE), lambda i: (i, 0)),
        ),
    )(ut, pt)


def _make_sc_gather():
    mesh = plsc.VectorSubcoreMesh(core_axis_name="c", subcore_axis_name="s")

    @functools.partial(
        pl.kernel,
        out_type=(
            jax.ShapeDtypeStruct((B, WIDE), jnp.float32),
            jax.ShapeDtypeStruct((B, WIDE), jnp.float32),
        ),
        mesh=mesh,
        scratch_types=[
            pltpu.VMEM((B_PER_W,), jnp.int32),
            pltpu.VMEM((B_PER_W,), jnp.int32),
            pltpu.VMEM((B_PER_W,), jnp.int32),
            pltpu.VMEM((B_PER_W, WIDE), jnp.float32),
            pltpu.SemaphoreType.DMA,
        ],
    )
    def gather(ut_hbm, pt_hbm, ui_hbm, pi_hbm, uo_hbm, po_hbm,
               uidx_v, pidx_v, sidx_v, rows_v, sem):
        wid = lax.axis_index("s") * NC + lax.axis_index("c")
        base = wid * B_PER_W
        pltpu.sync_copy(ui_hbm.at[pl.ds(base, B_PER_W)], uidx_v)
        pltpu.sync_copy(pi_hbm.at[pl.ds(base, B_PER_W)], pidx_v)

        def run_table(idx_v, t_hbm, o_hbm):
            for i in range(B_PER_W // L):
                sl = pl.ds(i * L, L)
                v = idx_v[sl]
                sidx_v[sl] = (lax.shift_right_logical(v, SH_PANEL) * SEG
                              + lax.bitwise_and(v, SEG - 1))
            copies = []
            for j in range(N_CHUNKS):
                sl = pl.ds(j * IDX_CHUNK, IDX_CHUNK)
                copies.append(pltpu.async_copy(
                    t_hbm.at[sidx_v.at[sl]], rows_v.at[sl], sem))
            for c in copies:
                c.wait()
            pltpu.sync_copy(rows_v, o_hbm.at[pl.ds(base, B_PER_W)])

        run_table(uidx_v, ut_hbm, uo_hbm)
        run_table(pidx_v, pt_hbm, po_hbm)

    return gather


_sc_gather = _make_sc_gather()

BLK = 1024


def _mlp_body(uf_ref, pf_ref, uix_ref, pix_ref, w1u_ref, w1p_ref, b1_ref,
              w2_ref, b2_ref, w3_ref, b3_ref, o_ref):
    colgrp = lax.broadcasted_iota(jnp.int32, (1, WIDE), 1) // D
    usel = lax.bitwise_and(lax.shift_right_logical(uix_ref[...], SH_SEG), 3)
    psel = lax.bitwise_and(lax.shift_right_logical(pix_ref[...], SH_SEG), 3)
    um = jnp.where(colgrp == usel, uf_ref[...], 0.0)
    pm = jnp.where(colgrp == psel, pf_ref[...], 0.0)
    h1 = (jnp.dot(um, w1u_ref[...], preferred_element_type=jnp.float32)
          + jnp.dot(pm, w1p_ref[...], preferred_element_type=jnp.float32)
          + b1_ref[...])
    h1 = jnp.maximum(h1, 0.0)
    h2 = jnp.dot(h1, w2_ref[...], preferred_element_type=jnp.float32) + b2_ref[...]
    h2 = jnp.maximum(h2, 0.0)
    o_ref[...] = jnp.sum(h2 * w3_ref[...], axis=1) + b3_ref[0, 0]


def _mlp(uf, pf, uix, pix, w1u4, w1p4, b1, w2, b2, w3, b3):
    full = lambda i: (0, 0)
    return pl.pallas_call(
        _mlp_body,
        out_shape=jax.ShapeDtypeStruct((B,), jnp.float32),
        grid=(B // BLK,),
        in_specs=[
            pl.BlockSpec((BLK, WIDE), lambda i: (i, 0)),
            pl.BlockSpec((BLK, WIDE), lambda i: (i, 0)),
            pl.BlockSpec((BLK, 1), lambda i: (i, 0)),
            pl.BlockSpec((BLK, 1), lambda i: (i, 0)),
            pl.BlockSpec((WIDE, 64), full),
            pl.BlockSpec((WIDE, 64), full),
            pl.BlockSpec((1, 64), full),
            pl.BlockSpec((64, 32), full),
            pl.BlockSpec((1, 32), full),
            pl.BlockSpec((1, 32), full),
            pl.BlockSpec((1, 1), full),
        ],
        out_specs=pl.BlockSpec((BLK,), lambda i: (i,)),
    )(uf, pf, uix, pix, w1u4, w1p4, b1, w2, b2, w3, b3)


def kernel(user_tensor, product_tensor, user_table, product_table,
           W1, b1, W2, b2, W3, b3):
    uix = user_tensor.astype(jnp.int32)
    pix = product_tensor.astype(jnp.int32)
    u_wide, p_wide = _pack(user_table.T, product_table.T)
    uf, pf = _sc_gather(u_wide, p_wide, uix, pix)
    w1u = W1[:, :D].T
    w1p = W1[:, D:].T
    w1u4 = jnp.concatenate([w1u] * GROUPS, axis=0)
    w1p4 = jnp.concatenate([w1p] * GROUPS, axis=0)
    return _mlp(uf, pf, uix.reshape(B, 1), pix.reshape(B, 1),
                w1u4, w1p4, b1.reshape(1, 64), W2.T, b2.reshape(1, 32),
                W3.reshape(1, 32), b3.reshape(1, 1))


# speedup vs baseline: 4.1019x; 1.1321x over previous
import functools

import jax
import jax.numpy as jnp
from jax import lax
from jax.experimental import pallas as pl
from jax.experimental.pallas import tpu as pltpu
from jax.experimental.pallas import tpu_sc as plsc

B = 16384
D = 32
N_ROWS = 1000000
GROUPS = 4
WIDE = D * GROUPS
PANEL = 16384
SEG = PANEL // GROUPS
SH_PANEL = PANEL.bit_length() - 1
SH_SEG = SEG.bit_length() - 1
N_PANELS = -(-N_ROWS // PANEL)
N_WIDE = N_PANELS * SEG
NC = 2
NS = 16
NW = NC * NS
B_PER_W = B // NW
IDX_CHUNK = 128
N_CHUNKS = B_PER_W // IDX_CHUNK
L = 16


def _pack_body(u_ref, p_ref, uo_ref, po_ref):
    for src, dst in ((u_ref, uo_ref), (p_ref, po_ref)):
        x = src[...]
        stacked = jnp.concatenate(
            [x[:, g * SEG:(g + 1) * SEG] for g in range(GROUPS)], axis=0)
        dst[...] = jnp.transpose(stacked)


def _pack(ut, pt):
    return pl.pallas_call(
        _pack_body,
        out_shape=(
            jax.ShapeDtypeStruct((N_WIDE, WIDE), jnp.float32),
            jax.ShapeDtypeStruct((N_WIDE, WIDE), jnp.float32),
        ),
        grid=(N_PANELS,),
        in_specs=[
            pl.BlockSpec((D, PANEL), lambda i: (0, i)),
            pl.BlockSpec((D, PANEL), lambda i: (0, i)),
        ],
        out_specs=(
            pl.BlockSpec((SEG, WIDE), lambda i: (i, 0)),
            pl.BlockSpec((SEG, WIDE), lambda i: (i, 0)),
        ),
    )(ut, pt)


def _make_sc_gather():
    mesh = plsc.VectorSubcoreMesh(core_axis_name="c", subcore_axis_name="s")

    @functools.partial(
        pl.kernel,
        out_type=(
            jax.ShapeDtypeStruct((B, WIDE), jnp.float32),
            jax.ShapeDtypeStruct((B, WIDE), jnp.float32),
        ),
        mesh=mesh,
        scratch_types=[
            pltpu.VMEM((B_PER_W,), jnp.int32),
            pltpu.VMEM((B_PER_W,), jnp.int32),
            pltpu.VMEM((B_PER_W,), jnp.int32),
            pltpu.VMEM((B_PER_W, WIDE), jnp.float32),
            pltpu.SemaphoreType.DMA,
        ],
    )
    def gather(ut_hbm, pt_hbm, ui_hbm, pi_hbm, uo_hbm, po_hbm,
               uidx_v, pidx_v, sidx_v, rows_v, sem):
        wid = lax.axis_index("s") * NC + lax.axis_index("c")
        base = wid * B_PER_W
        pltpu.sync_copy(ui_hbm.at[pl.ds(base, B_PER_W)], uidx_v)
        pltpu.sync_copy(pi_hbm.at[pl.ds(base, B_PER_W)], pidx_v)

        def run_table(idx_v, t_hbm, o_hbm):
            for i in range(B_PER_W // L):
                sl = pl.ds(i * L, L)
                v = idx_v[sl]
                sidx_v[sl] = (lax.shift_right_logical(v, SH_PANEL) * SEG
                              + lax.bitwise_and(v, SEG - 1))
            copies = []
            for j in range(N_CHUNKS):
                sl = pl.ds(j * IDX_CHUNK, IDX_CHUNK)
                copies.append(pltpu.async_copy(
                    t_hbm.at[sidx_v.at[sl]], rows_v.at[sl], sem))
            for c in copies:
                c.wait()
            pltpu.sync_copy(rows_v, o_hbm.at[pl.ds(base, B_PER_W)])

        run_table(uidx_v, ut_hbm, uo_hbm)
        run_table(pidx_v, pt_hbm, po_hbm)

    return gather


_sc_gather = _make_sc_gather()

BLK = 1024


def _mlp_body(uf_ref, pf_ref, uix_ref, pix_ref, w1u_ref, w1p_ref, b1_ref,
              w2_ref, b2_ref, w3_ref, b3_ref, o_ref):
    colgrp = lax.broadcasted_iota(jnp.int32, (1, WIDE), 1) // D
    usel = lax.bitwise_and(lax.shift_right_logical(uix_ref[...], SH_SEG), 3)
    psel = lax.bitwise_and(lax.shift_right_logical(pix_ref[...], SH_SEG), 3)
    um = jnp.where(colgrp == usel, uf_ref[...], 0.0)
    pm = jnp.where(colgrp == psel, pf_ref[...], 0.0)
    h1 = (jnp.dot(um, w1u_ref[...], preferred_element_type=jnp.float32)
          + jnp.dot(pm, w1p_ref[...], preferred_element_type=jnp.float32)
          + b1_ref[...])
    h1 = jnp.maximum(h1, 0.0)
    h2 = jnp.dot(h1, w2_ref[...], preferred_element_type=jnp.float32) + b2_ref[...]
    h2 = jnp.maximum(h2, 0.0)
    o_ref[...] = jnp.sum(h2 * w3_ref[...], axis=1) + b3_ref[0, 0]


def _mlp(uf, pf, uix, pix, w1u4, w1p4, b1, w2, b2, w3, b3):
    full = lambda i: (0, 0)
    return pl.pallas_call(
        _mlp_body,
        out_shape=jax.ShapeDtypeStruct((B,), jnp.float32),
        grid=(B // BLK,),
        in_specs=[
            pl.BlockSpec((BLK, WIDE), lambda i: (i, 0)),
            pl.BlockSpec((BLK, WIDE), lambda i: (i, 0)),
            pl.BlockSpec((BLK, 1), lambda i: (i, 0)),
            pl.BlockSpec((BLK, 1), lambda i: (i, 0)),
            pl.BlockSpec((WIDE, 64), full),
            pl.BlockSpec((WIDE, 64), full),
            pl.BlockSpec((1, 64), full),
            pl.BlockSpec((64, 32), full),
            pl.BlockSpec((1, 32), full),
            pl.BlockSpec((1, 32), full),
            pl.BlockSpec((1, 1), full),
        ],
        out_specs=pl.BlockSpec((BLK,), lambda i: (i,)),
    )(uf, pf, uix, pix, w1u4, w1p4, b1, w2, b2, w3, b3)


def kernel(user_tensor, product_tensor, user_table, product_table,
           W1, b1, W2, b2, W3, b3):
    uix = user_tensor.astype(jnp.int32)
    pix = product_tensor.astype(jnp.int32)
    u_wide, p_wide = _pack(user_table.T, product_table.T)
    uf, pf = _sc_gather(u_wide, p_wide, uix, pix)
    w1u = W1[:, :D].T
    w1p = W1[:, D:].T
    w1u4 = jnp.concatenate([w1u] * GROUPS, axis=0)
    w1p4 = jnp.concatenate([w1p] * GROUPS, axis=0)
    return _mlp(uf, pf, uix.reshape(B, 1), pix.reshape(B, 1),
                w1u4, w1p4, b1.reshape(1, 64), W2.T, b2.reshape(1, 32),
                W3.reshape(1, 32), b3.reshape(1, 1))


# speedup vs baseline: 4.1967x; 1.0231x over previous
import functools

import jax
import jax.numpy as jnp
from jax import lax
from jax.experimental import pallas as pl
from jax.experimental.pallas import tpu as pltpu
from jax.experimental.pallas import tpu_sc as plsc

B = 16384
D = 32
N_ROWS = 1000000
GROUPS = 4
WIDE = D * GROUPS
PANEL = 32768
SEG = PANEL // GROUPS
SH_PANEL = PANEL.bit_length() - 1
SH_SEG = SEG.bit_length() - 1
N_PANELS = -(-N_ROWS // PANEL)
N_WIDE = N_PANELS * SEG
NC = 2
NS = 16
NW = NC * NS
B_PER_W = B // NW
IDX_CHUNK = 128
N_CHUNKS = B_PER_W // IDX_CHUNK
L = 16


def _pack_body(u_ref, p_ref, uo_ref, po_ref):
    for src, dst in ((u_ref, uo_ref), (p_ref, po_ref)):
        x = src[...]
        stacked = jnp.concatenate(
            [x[:, g * SEG:(g + 1) * SEG] for g in range(GROUPS)], axis=0)
        dst[...] = jnp.transpose(stacked)


def _pack(ut, pt):
    return pl.pallas_call(
        _pack_body,
        out_shape=(
            jax.ShapeDtypeStruct((N_WIDE, WIDE), jnp.float32),
            jax.ShapeDtypeStruct((N_WIDE, WIDE), jnp.float32),
        ),
        grid=(N_PANELS,),
        in_specs=[
            pl.BlockSpec((D, PANEL), lambda i: (0, i)),
            pl.BlockSpec((D, PANEL), lambda i: (0, i)),
        ],
        out_specs=(
            pl.BlockSpec((SEG, WIDE), lambda i: (i, 0)),
            pl.BlockSpec((SEG, WIDE), lambda i: (i, 0)),
        ),
    )(ut, pt)


def _make_sc_gather():
    mesh = plsc.VectorSubcoreMesh(core_axis_name="c", subcore_axis_name="s")

    @functools.partial(
        pl.kernel,
        out_type=(
            jax.ShapeDtypeStruct((B, WIDE), jnp.float32),
            jax.ShapeDtypeStruct((B, WIDE), jnp.float32),
        ),
        mesh=mesh,
        scratch_types=[
            pltpu.VMEM((B_PER_W,), jnp.int32),
            pltpu.VMEM((B_PER_W,), jnp.int32),
            pltpu.VMEM((B_PER_W,), jnp.int32),
            pltpu.VMEM((B_PER_W, WIDE), jnp.float32),
            pltpu.SemaphoreType.DMA,
        ],
    )
    def gather(ut_hbm, pt_hbm, ui_hbm, pi_hbm, uo_hbm, po_hbm,
               uidx_v, pidx_v, sidx_v, rows_v, sem):
        wid = lax.axis_index("s") * NC + lax.axis_index("c")
        base = wid * B_PER_W
        pltpu.sync_copy(ui_hbm.at[pl.ds(base, B_PER_W)], uidx_v)
        pltpu.sync_copy(pi_hbm.at[pl.ds(base, B_PER_W)], pidx_v)

        def run_table(idx_v, t_hbm, o_hbm):
            for i in range(B_PER_W // L):
                sl = pl.ds(i * L, L)
                v = idx_v[sl]
                sidx_v[sl] = (lax.shift_right_logical(v, SH_PANEL) * SEG
                              + lax.bitwise_and(v, SEG - 1))
            copies = []
            for j in range(N_CHUNKS):
                sl = pl.ds(j * IDX_CHUNK, IDX_CHUNK)
                copies.append(pltpu.async_copy(
                    t_hbm.at[sidx_v.at[sl]], rows_v.at[sl], sem))
            for c in copies:
                c.wait()
            pltpu.sync_copy(rows_v, o_hbm.at[pl.ds(base, B_PER_W)])

        run_table(uidx_v, ut_hbm, uo_hbm)
        run_table(pidx_v, pt_hbm, po_hbm)

    return gather


_sc_gather = _make_sc_gather()

BLK = 1024


def _mlp_body(uf_ref, pf_ref, uix_ref, pix_ref, w1u_ref, w1p_ref, b1_ref,
              w2_ref, b2_ref, w3_ref, b3_ref, o_ref):
    colgrp = lax.broadcasted_iota(jnp.int32, (1, WIDE), 1) // D
    usel = lax.bitwise_and(lax.shift_right_logical(uix_ref[...], SH_SEG), 3)
    psel = lax.bitwise_and(lax.shift_right_logical(pix_ref[...], SH_SEG), 3)
    um = jnp.where(colgrp == usel, uf_ref[...], 0.0)
    pm = jnp.where(colgrp == psel, pf_ref[...], 0.0)
    h1 = (jnp.dot(um, w1u_ref[...], preferred_element_type=jnp.float32)
          + jnp.dot(pm, w1p_ref[...], preferred_element_type=jnp.float32)
          + b1_ref[...])
    h1 = jnp.maximum(h1, 0.0)
    h2 = jnp.dot(h1, w2_ref[...], preferred_element_type=jnp.float32) + b2_ref[...]
    h2 = jnp.maximum(h2, 0.0)
    o_ref[...] = jnp.sum(h2 * w3_ref[...], axis=1) + b3_ref[0, 0]


def _mlp(uf, pf, uix, pix, w1u4, w1p4, b1, w2, b2, w3, b3):
    full = lambda i: (0, 0)
    return pl.pallas_call(
        _mlp_body,
        out_shape=jax.ShapeDtypeStruct((B,), jnp.float32),
        grid=(B // BLK,),
        in_specs=[
            pl.BlockSpec((BLK, WIDE), lambda i: (i, 0)),
            pl.BlockSpec((BLK, WIDE), lambda i: (i, 0)),
            pl.BlockSpec((BLK, 1), lambda i: (i, 0)),
            pl.BlockSpec((BLK, 1), lambda i: (i, 0)),
            pl.BlockSpec((WIDE, 64), full),
            pl.BlockSpec((WIDE, 64), full),
            pl.BlockSpec((1, 64), full),
            pl.BlockSpec((64, 32), full),
            pl.BlockSpec((1, 32), full),
            pl.BlockSpec((1, 32), full),
            pl.BlockSpec((1, 1), full),
        ],
        out_specs=pl.BlockSpec((BLK,), lambda i: (i,)),
    )(uf, pf, uix, pix, w1u4, w1p4, b1, w2, b2, w3, b3)


def kernel(user_tensor, product_tensor, user_table, product_table,
           W1, b1, W2, b2, W3, b3):
    uix = user_tensor.astype(jnp.int32)
    pix = product_tensor.astype(jnp.int32)
    u_wide, p_wide = _pack(user_table.T, product_table.T)
    uf, pf = _sc_gather(u_wide, p_wide, uix, pix)
    w1u = W1[:, :D].T
    w1p = W1[:, D:].T
    w1u4 = jnp.concatenate([w1u] * GROUPS, axis=0)
    w1p4 = jnp.concatenate([w1p] * GROUPS, axis=0)
    return _mlp(uf, pf, uix.reshape(B, 1), pix.reshape(B, 1),
                w1u4, w1p4, b1.reshape(1, 64), W2.T, b2.reshape(1, 32),
                W3.reshape(1, 32), b3.reshape(1, 1))


# speedup vs baseline: 4.3726x; 1.0419x over previous
import functools

import jax
import jax.numpy as jnp
from jax import lax
from jax.experimental import pallas as pl
from jax.experimental.pallas import tpu as pltpu
from jax.experimental.pallas import tpu_sc as plsc

B = 16384
D = 32
N_ROWS = 1000000
GROUPS = 4
WIDE = D * GROUPS
PANEL = 32768
SEG = PANEL // GROUPS
SH_PANEL = PANEL.bit_length() - 1
SH_SEG = SEG.bit_length() - 1
N_PANELS = -(-N_ROWS // PANEL)
N_WIDE = N_PANELS * SEG
NC = 2
NS = 16
NW = NC * NS
B_PER_W = B // NW
IDX_CHUNK = 128
N_CHUNKS = B_PER_W // IDX_CHUNK
L = 16


def _pack_body(u_ref, p_ref, uo_ref, po_ref):
    for src, dst in ((u_ref, uo_ref), (p_ref, po_ref)):
        x = src[...]
        stacked = jnp.concatenate(
            [x[:, g * SEG:(g + 1) * SEG] for g in range(GROUPS)], axis=0)
        dst[...] = jnp.transpose(stacked)


def _pack(ut, pt):
    return pl.pallas_call(
        _pack_body,
        out_shape=(
            jax.ShapeDtypeStruct((N_WIDE, WIDE), jnp.float32),
            jax.ShapeDtypeStruct((N_WIDE, WIDE), jnp.float32),
        ),
        grid=(N_PANELS,),
        in_specs=[
            pl.BlockSpec((D, PANEL), lambda i: (0, i)),
            pl.BlockSpec((D, PANEL), lambda i: (0, i)),
        ],
        out_specs=(
            pl.BlockSpec((SEG, WIDE), lambda i: (i, 0)),
            pl.BlockSpec((SEG, WIDE), lambda i: (i, 0)),
        ),
    )(ut, pt)


def _make_sc_gather():
    mesh = plsc.VectorSubcoreMesh(core_axis_name="c", subcore_axis_name="s")

    @functools.partial(
        pl.kernel,
        out_type=(
            jax.ShapeDtypeStruct((B, WIDE), jnp.float32),
            jax.ShapeDtypeStruct((B, WIDE), jnp.float32),
        ),
        mesh=mesh,
        scratch_types=[
            pltpu.VMEM((B_PER_W,), jnp.int32),
            pltpu.VMEM((B_PER_W,), jnp.int32),
            pltpu.VMEM((B_PER_W,), jnp.int32),
            pltpu.VMEM((B_PER_W, WIDE), jnp.float32),
            pltpu.SemaphoreType.DMA,
        ],
    )
    def gather(ut_hbm, pt_hbm, ui_hbm, pi_hbm, uo_hbm, po_hbm,
               uidx_v, pidx_v, sidx_v, rows_v, sem):
        wid = lax.axis_index("s") * NC + lax.axis_index("c")
        base = wid * B_PER_W
        pltpu.sync_copy(ui_hbm.at[pl.ds(base, B_PER_W)], uidx_v)
        pltpu.sync_copy(pi_hbm.at[pl.ds(base, B_PER_W)], pidx_v)

        def run_table(idx_v, t_hbm, o_hbm):
            for i in range(B_PER_W // L):
                sl = pl.ds(i * L, L)
                v = idx_v[sl]
                sidx_v[sl] = (lax.shift_right_logical(v, SH_PANEL) * SEG
                              + lax.bitwise_and(v, SEG - 1))
            copies = []
            for j in range(N_CHUNKS):
                sl = pl.ds(j * IDX_CHUNK, IDX_CHUNK)
                copies.append(pltpu.async_copy(
                    t_hbm.at[sidx_v.at[sl]], rows_v.at[sl], sem))
            for c in copies:
                c.wait()
            pltpu.sync_copy(rows_v, o_hbm.at[pl.ds(base, B_PER_W)])

        run_table(uidx_v, ut_hbm, uo_hbm)
        run_table(pidx_v, pt_hbm, po_hbm)

    return gather


_sc_gather = _make_sc_gather()

BLK = 4096


def _mlp_body(uf_ref, pf_ref, uix_ref, pix_ref, w1u_ref, w1p_ref, b1_ref,
              w2_ref, b2_ref, w3_ref, b3_ref, o_ref):
    colgrp = lax.broadcasted_iota(jnp.int32, (1, WIDE), 1) // D
    usel = lax.bitwise_and(lax.shift_right_logical(uix_ref[...], SH_SEG), 3)
    psel = lax.bitwise_and(lax.shift_right_logical(pix_ref[...], SH_SEG), 3)
    um = jnp.where(colgrp == usel, uf_ref[...], 0.0)
    pm = jnp.where(colgrp == psel, pf_ref[...], 0.0)
    h1 = (jnp.dot(um, w1u_ref[...], preferred_element_type=jnp.float32)
          + jnp.dot(pm, w1p_ref[...], preferred_element_type=jnp.float32)
          + b1_ref[...])
    h1 = jnp.maximum(h1, 0.0)
    h2 = jnp.dot(h1, w2_ref[...], preferred_element_type=jnp.float32) + b2_ref[...]
    h2 = jnp.maximum(h2, 0.0)
    o_ref[...] = jnp.sum(h2 * w3_ref[...], axis=1) + b3_ref[0, 0]


def _mlp(uf, pf, uix, pix, w1u4, w1p4, b1, w2, b2, w3, b3):
    full = lambda i: (0, 0)
    return pl.pallas_call(
        _mlp_body,
        out_shape=jax.ShapeDtypeStruct((B,), jnp.float32),
        grid=(B // BLK,),
        in_specs=[
            pl.BlockSpec((BLK, WIDE), lambda i: (i, 0)),
            pl.BlockSpec((BLK, WIDE), lambda i: (i, 0)),
            pl.BlockSpec((BLK, 1), lambda i: (i, 0)),
            pl.BlockSpec((BLK, 1), lambda i: (i, 0)),
            pl.BlockSpec((WIDE, 64), full),
            pl.BlockSpec((WIDE, 64), full),
            pl.BlockSpec((1, 64), full),
            pl.BlockSpec((64, 32), full),
            pl.BlockSpec((1, 32), full),
            pl.BlockSpec((1, 32), full),
            pl.BlockSpec((1, 1), full),
        ],
        out_specs=pl.BlockSpec((BLK,), lambda i: (i,)),
    )(uf, pf, uix, pix, w1u4, w1p4, b1, w2, b2, w3, b3)


def kernel(user_tensor, product_tensor, user_table, product_table,
           W1, b1, W2, b2, W3, b3):
    uix = user_tensor.astype(jnp.int32)
    pix = product_tensor.astype(jnp.int32)
    u_wide, p_wide = _pack(user_table.T, product_table.T)
    uf, pf = _sc_gather(u_wide, p_wide, uix, pix)
    w1u = W1[:, :D].T
    w1p = W1[:, D:].T
    w1u4 = jnp.concatenate([w1u] * GROUPS, axis=0)
    w1p4 = jnp.concatenate([w1p] * GROUPS, axis=0)
    return _mlp(uf, pf, uix.reshape(B, 1), pix.reshape(B, 1),
                w1u4, w1p4, b1.reshape(1, 64), W2.T, b2.reshape(1, 32),
                W3.reshape(1, 32), b3.reshape(1, 1))
